# Initial kernel scaffold; baseline (speedup 1.0000x reference)
import jax
import jax.numpy as jnp
from jax.experimental import pallas as pl


def kernel(x, edge_index, edge_type, batch, W1, root1, b1, W2, root2, b2, Wm1, bm1, Wm2, bm2):
    raise NotImplementedError("write your pallas kernel here")


# speedup vs baseline: 10.1204x; 10.1204x over previous
import dataclasses
import functools

import jax
import jax.numpy as jnp
from jax import lax
from jax.experimental import pallas as pl
from jax.experimental.pallas import tpu as pltpu
from jax.experimental.pallas import tpu_sc as plsc

N = 10000
E = 320000
NREL = 8
G = 64
D = 128
RK = NREL + 1
MLP = 256
NCLS = 10

NC = 2
NS = 16
NW = NC * NS
L = 16

EPT = 10240
EP = NW * EPT
EROWS = EP // 128
TBL = 80128
PAD_BIN = 80000
NACC = 10240
NPOOL = 12288
PB = 80


def _sc_mesh():
    return plsc.VectorSubcoreMesh(core_axis_name="c", subcore_axis_name="s")


def _sc_params():
    cp = pltpu.CompilerParams()
    if "needs_layout_passes" in pltpu.CompilerParams.__dataclass_fields__:
        cp = dataclasses.replace(cp, needs_layout_passes=False)
    return cp


def _wid():
    return lax.axis_index("s") * NC + lax.axis_index("c")


def _count_body(kd_hbm, cnt_hbm, tbl_v, kd_v):
    w = _wid()
    z16 = jnp.zeros((L,), jnp.float32)
    ones = jnp.ones((L,), jnp.float32)

    @pl.loop(0, TBL // L)
    def _(i):
        tbl_v[pl.ds(i * L, L)] = z16

    pltpu.sync_copy(kd_hbm.at[pl.ds(w * 80, 80)], kd_v)

    @pl.loop(0, 80)
    def _(j):
        for k in range(8):
            kk = kd_v[j, pl.ds(k * L, L)]
            plsc.addupdate_scatter(tbl_v, [kk], ones)

    pltpu.sync_copy(tbl_v, cnt_hbm.at[pl.ds(w * TBL, TBL)])


def _counts(kd2d):
    kern = pl.kernel(
        _count_body,
        out_type=jax.ShapeDtypeStruct((NW * TBL,), jnp.float32),
        mesh=_sc_mesh(),
        compiler_params=_sc_params(),
        scratch_types=[
            pltpu.VMEM((TBL,), jnp.float32),
            pltpu.VMEM((80, 128), jnp.int32),
        ],
    )
    return kern(kd2d)


def _scale_body(inv_hbm, kd_hbm, w_hbm, inv_v, kd_v, wout_v):
    w = _wid()
    pltpu.sync_copy(inv_hbm, inv_v)
    pltpu.sync_copy(kd_hbm.at[pl.ds(w * 80, 80)], kd_v)

    @pl.loop(0, 80)
    def _(j):
        for k in range(8):
            kk = kd_v[j, pl.ds(k * L, L)]
            wout_v[j, pl.ds(k * L, L)] = plsc.load_gather(inv_v, [kk])

    pltpu.sync_copy(wout_v, w_hbm.at[pl.ds(w * 80, 80)])


def _scales(inv, kd2d):
    kern = pl.kernel(
        _scale_body,
        out_type=jax.ShapeDtypeStruct((EROWS, 128), jnp.float32),
        mesh=_sc_mesh(),
        compiler_params=_sc_params(),
        scratch_types=[
            pltpu.VMEM((TBL,), jnp.float32),
            pltpu.VMEM((80, 128), jnp.int32),
            pltpu.VMEM((80, 128), jnp.float32),
        ],
    )
    return kern(inv, kd2d)


def _scat_body(h_hbm, ks_hbm, dst_hbm, w_hbm, acc_hbm,
               ks_v, dst_v, w_v, rows_v, acc_sh, sem):
    cid = lax.axis_index("c")
    sid = lax.axis_index("s")
    w = sid * NC + cid
    z16 = jnp.zeros((L,), jnp.float32)

    @pl.loop(0, 128)
    def _(i):
        for c in range(8):
            rows_v[i, pl.ds(c * L, L)] = z16

    for t in range(5):
        pltpu.sync_copy(rows_v, acc_sh.at[pl.ds(sid * 640 + t * 128, 128)])
    plsc.subcore_barrier()

    pltpu.sync_copy(ks_hbm.at[pl.ds(w * 80, 80)], ks_v)
    pltpu.sync_copy(dst_hbm.at[pl.ds(w * 80, 80)], dst_v)
    pltpu.sync_copy(w_hbm.at[pl.ds(w * 80, 80)], w_v)

    @pl.loop(0, 80)
    def _(j):
        pltpu.async_copy(h_hbm.at[ks_v.at[j]], rows_v, sem).wait()

        @pl.loop(0, 128)
        def _(e):
            j16 = jnp.full((L,), j, jnp.int32)
            e16 = jnp.full((L,), e, jnp.int32)
            w16 = plsc.load_gather(w_v, [j16, e16])
            for c in range(8):
                rows_v[e, pl.ds(c * L, L)] = rows_v[e, pl.ds(c * L, L)] * w16

        pltpu.sync_copy(rows_v, acc_sh.at[dst_v.at[j]], add=True)

    plsc.subcore_barrier()
    for t in range(5):
        r0 = sid * 640 + t * 128
        pltpu.sync_copy(acc_sh.at[pl.ds(r0, 128)],
                        acc_hbm.at[pl.ds(cid * NACC + r0, 128)])


def _scatter_layer(h_flat, ks2d, dst2d, w2d):
    kern = pl.kernel(
        _scat_body,
        out_type=jax.ShapeDtypeStruct((NC * NACC, D), jnp.float32),
        mesh=_sc_mesh(),
        compiler_params=_sc_params(),
        scratch_types=[
            pltpu.VMEM((80, 128), jnp.int32),
            pltpu.VMEM((80, 128), jnp.int32),
            pltpu.VMEM((80, 128), jnp.float32),
            pltpu.VMEM((128, D), jnp.float32),
            pltpu.VMEM_SHARED((NACC, D), jnp.float32),
            pltpu.SemaphoreType.DMA,
        ],
    )
    return kern(h_flat, ks2d, dst2d, w2d)


def _pool_body(h_hbm, b_hbm, pool_hbm, cnt_hbm, rows_v, bidx_v, cnt_v, acc_sh):
    cid = lax.axis_index("c")
    sid = lax.axis_index("s")
    w = sid * NC + cid
    z16 = jnp.zeros((L,), jnp.float32)
    ones = jnp.ones((L,), jnp.float32)

    @pl.loop(0, PB // L)
    def _(i):
        cnt_v[pl.ds(i * L, L)] = z16

    @pl.loop(0, 128)
    def _(i):
        for c in range(8):
            rows_v[i, pl.ds(c * L, L)] = z16

    @pl.when(sid == 0)
    def _():
        pltpu.sync_copy(rows_v.at[pl.ds(0, PB)], acc_sh)

    plsc.subcore_barrier()

    pltpu.sync_copy(b_hbm.at[w], bidx_v)
    for j in range(3):
        pltpu.sync_copy(h_hbm.at[pl.ds(w * 384 + j * 128, 128)], rows_v)
        pltpu.sync_copy(rows_v, acc_sh.at[bidx_v.at[j]], add=True)
        for k in range(8):
            kb = bidx_v[j, pl.ds(k * L, L)]
            plsc.addupdate_scatter(cnt_v, [kb], ones)

    plsc.subcore_barrier()

    @pl.when(sid == 0)
    def _():
        pltpu.sync_copy(acc_sh, pool_hbm.at[pl.ds(cid * PB, PB)])

    pltpu.sync_copy(cnt_v, cnt_hbm.at[pl.ds(w * PB, PB)])


def _pool(h2p, batch3d):
    kern = pl.kernel(
        _pool_body,
        out_type=(jax.ShapeDtypeStruct((NC * PB, D), jnp.float32),
                  jax.ShapeDtypeStruct((NW * PB,), jnp.float32)),
        mesh=_sc_mesh(),
        compiler_params=_sc_params(),
        scratch_types=[
            pltpu.VMEM((128, D), jnp.float32),
            pltpu.VMEM((3, 128), jnp.int32),
            pltpu.VMEM((PB,), jnp.float32),
            pltpu.VMEM_SHARED((PB, D), jnp.float32),
        ],
    )
    return kern(h2p, batch3d)


def _mm_body(x_ref, w_ref, o_ref):
    o_ref[...] = jnp.dot(x_ref[...], w_ref[...],
                         preferred_element_type=jnp.float32)


def _matmul1(x, wcat):
    return pl.pallas_call(
        _mm_body,
        grid=(10,),
        in_specs=[pl.BlockSpec((1000, D), lambda i: (i, 0)),
                  pl.BlockSpec((D, RK * D), lambda i: (0, 0))],
        out_specs=pl.BlockSpec((1000, RK * D), lambda i: (i, 0)),
        out_shape=jax.ShapeDtypeStruct((N, RK * D), jnp.float32),
    )(x, wcat)


def _comb_mm_body(a0_ref, a1_ref, hr_ref, b_ref, w_ref, o_ref):
    h = a0_ref[...] + a1_ref[...] + hr_ref[...] + b_ref[...]
    h = jnp.maximum(h, 0.0)
    o_ref[...] = jnp.dot(h, w_ref[...], preferred_element_type=jnp.float32)


def _comb_matmul(acc0, acc1, hroot, b, wcat):
    return pl.pallas_call(
        _comb_mm_body,
        grid=(10,),
        in_specs=[pl.BlockSpec((1000, D), lambda i: (i, 0)),
                  pl.BlockSpec((1000, D), lambda i: (i, 0)),
                  pl.BlockSpec((1000, D), lambda i: (i, 0)),
                  pl.BlockSpec((1, D), lambda i: (0, 0)),
                  pl.BlockSpec((D, RK * D), lambda i: (0, 0))],
        out_specs=pl.BlockSpec((1000, RK * D), lambda i: (i, 0)),
        out_shape=jax.ShapeDtypeStruct((N, RK * D), jnp.float32),
    )(acc0, acc1, hroot, b, wcat)


def _comb_body(a0_ref, a1_ref, hr_ref, b_ref, o_ref):
    h = a0_ref[...] + a1_ref[...] + hr_ref[...] + b_ref[...]
    o_ref[...] = jnp.maximum(h, 0.0)


def _comb_relu(acc0, acc1, hroot, b):
    return pl.pallas_call(
        _comb_body,
        grid=(10,),
        in_specs=[pl.BlockSpec((1000, D), lambda i: (i, 0)),
                  pl.BlockSpec((1000, D), lambda i: (i, 0)),
                  pl.BlockSpec((1000, D), lambda i: (i, 0)),
                  pl.BlockSpec((1, D), lambda i: (0, 0))],
        out_specs=pl.BlockSpec((1000, D), lambda i: (i, 0)),
        out_shape=jax.ShapeDtypeStruct((NPOOL, D), jnp.float32),
    )(acc0, acc1, hroot, b)


def _inv_body(c_ref, o_ref):
    s = jnp.sum(c_ref[...], axis=0)
    gid = (128 * lax.broadcasted_iota(jnp.int32, (TBL // 128, 128), 0)
           + lax.broadcasted_iota(jnp.int32, (TBL // 128, 128), 1))
    o_ref[...] = jnp.where(gid < PAD_BIN, 1.0 / jnp.maximum(s, 1.0), 0.0)


def _inv_counts(cnt32):
    c3 = cnt32.reshape(NW, TBL // 128, 128)
    out = pl.pallas_call(
        _inv_body,
        out_shape=jax.ShapeDtypeStruct((TBL // 128, 128), jnp.float32),
    )(c3)
    return out.reshape(TBL)


def _mlp_body(p_ref, c_ref, w1_ref, b1_ref, w2_ref, b2_ref, o_ref):
    cnt = jnp.sum(c_ref[...], axis=0)[:G]
    g = p_ref[0:G, :] + p_ref[PB:PB + G, :]
    g = g / jnp.maximum(cnt, 1.0)[:, None]
    a = jnp.maximum(jnp.dot(g, w1_ref[...],
                            preferred_element_type=jnp.float32)
                    + b1_ref[...], 0.0)
    logits = jnp.dot(a, w2_ref[...],
                     preferred_element_type=jnp.float32) + b2_ref[...]
    m = jnp.max(logits, axis=1, keepdims=True)
    s = logits - m
    o_ref[...] = s - jnp.log(jnp.sum(jnp.exp(s), axis=1, keepdims=True))


def _mlp_head(pool2, cntb, wm1, bm1, wm2, bm2):
    return pl.pallas_call(
        _mlp_body,
        in_specs=[pl.BlockSpec((NC * PB, D), lambda: (0, 0)),
                  pl.BlockSpec((NW, PB), lambda: (0, 0)),
                  pl.BlockSpec((D, MLP), lambda: (0, 0)),
                  pl.BlockSpec((1, MLP), lambda: (0, 0)),
                  pl.BlockSpec((MLP, NCLS), lambda: (0, 0)),
                  pl.BlockSpec((1, NCLS), lambda: (0, 0))],
        out_specs=pl.BlockSpec((G, NCLS), lambda: (0, 0)),
        out_shape=jax.ShapeDtypeStruct((G, NCLS), jnp.float32),
    )(pool2, cntb, wm1, bm1, wm2, bm2)


def kernel(x, edge_index, edge_type, batch, W1, root1, b1, W2, root2, b2,
           Wm1, bm1, Wm2, bm2):
    f32, i32 = jnp.float32, jnp.int32
    src = edge_index[0]
    dst = edge_index[1]
    et = edge_type

    pad = EP - E
    ks2d = jnp.concatenate(
        [src * RK + et, jnp.zeros((pad,), i32)]).reshape(EROWS, 128)
    kd2d = jnp.concatenate(
        [dst * NREL + et, jnp.full((pad,), PAD_BIN, i32)]).reshape(EROWS, 128)
    dst2d = jnp.concatenate(
        [dst, jnp.zeros((pad,), i32)]).reshape(EROWS, 128)
    batch3d = jnp.concatenate(
        [batch, jnp.full((NPOOL - N,), G, i32)]).reshape(NW, 3, 128)

    wcat1 = jnp.concatenate([W1.transpose(1, 0, 2).reshape(D, NREL * D),
                             root1], axis=1)
    wcat2 = jnp.concatenate([W2.transpose(1, 0, 2).reshape(D, NREL * D),
                             root2], axis=1)

    cnt32 = _counts(kd2d).reshape(NW, TBL)
    inv = _inv_counts(cnt32)
    w2d = _scales(inv, kd2d)

    h1cat = _matmul1(x, wcat1)
    acc1 = _scatter_layer(h1cat.reshape(RK * N, D), ks2d, dst2d, w2d)
    acc1 = acc1.reshape(NC, NACC, D)

    h2cat = _comb_matmul(acc1[0, :N], acc1[1, :N],
                         h1cat[:, NREL * D:], b1.reshape(1, D), wcat2)
    acc2 = _scatter_layer(h2cat.reshape(RK * N, D), ks2d, dst2d, w2d)
    acc2 = acc2.reshape(NC, NACC, D)

    h2p = _comb_relu(acc2[0, :N], acc2[1, :N],
                     h2cat[:, NREL * D:], b2.reshape(1, D))

    pool2, cntb = _pool(h2p, batch3d)
    cntb = cntb.reshape(NW, PB)
    return _mlp_head(pool2, cntb, Wm1, bm1.reshape(1, MLP),
                     Wm2, bm2.reshape(1, NCLS))


# speedup vs baseline: 12.2618x; 1.2116x over previous
import dataclasses
import functools

import jax
import jax.numpy as jnp
from jax import lax
from jax.experimental import pallas as pl
from jax.experimental.pallas import tpu as pltpu
from jax.experimental.pallas import tpu_sc as plsc

N = 10000
E = 320000
NREL = 8
G = 64
D = 128
RK = NREL + 1
MLP = 256
NCLS = 10

NC = 2
NS = 16
NW = NC * NS
L = 16

EPT = 10240
EP = NW * EPT
EROWS = EP // 128
TBL = 80128
PAD_BIN = 80000
NACC = 10240
NPOOL = 12288
PB = 80


def _sc_mesh():
    return plsc.VectorSubcoreMesh(core_axis_name="c", subcore_axis_name="s")


def _sc_params():
    cp = pltpu.CompilerParams()
    if "needs_layout_passes" in pltpu.CompilerParams.__dataclass_fields__:
        cp = dataclasses.replace(cp, needs_layout_passes=False)
    return cp


def _wid():
    return lax.axis_index("s") * NC + lax.axis_index("c")


def _count_body(kd_hbm, cnt_hbm, tbl_v, kd_v):
    w = _wid()
    z16 = jnp.zeros((L,), jnp.float32)
    ones = jnp.ones((L,), jnp.float32)

    @pl.loop(0, TBL // L)
    def _(i):
        tbl_v[pl.ds(i * L, L)] = z16

    pltpu.sync_copy(kd_hbm.at[pl.ds(w * 80, 80)], kd_v)

    @pl.loop(0, 80)
    def _(j):
        for k in range(8):
            kk = kd_v[j, pl.ds(k * L, L)]
            plsc.addupdate_scatter(tbl_v, [kk], ones)

    pltpu.sync_copy(tbl_v, cnt_hbm.at[pl.ds(w * TBL, TBL)])


def _counts(kd2d):
    kern = pl.kernel(
        _count_body,
        out_type=jax.ShapeDtypeStruct((NW * TBL,), jnp.float32),
        mesh=_sc_mesh(),
        compiler_params=_sc_params(),
        scratch_types=[
            pltpu.VMEM((TBL,), jnp.float32),
            pltpu.VMEM((80, 128), jnp.int32),
        ],
    )
    return kern(kd2d)


def _scale_body(inv_hbm, kd_hbm, w_hbm, inv_v, kd_v, wout_v):
    w = _wid()
    pltpu.sync_copy(inv_hbm, inv_v)
    pltpu.sync_copy(kd_hbm.at[pl.ds(w * 80, 80)], kd_v)

    @pl.loop(0, 80)
    def _(j):
        for k in range(8):
            kk = kd_v[j, pl.ds(k * L, L)]
            wout_v[j, pl.ds(k * L, L)] = plsc.load_gather(inv_v, [kk])

    pltpu.sync_copy(wout_v, w_hbm.at[pl.ds(w * 80, 80)])


def _scales(inv, kd2d):
    kern = pl.kernel(
        _scale_body,
        out_type=jax.ShapeDtypeStruct((EROWS, 128), jnp.float32),
        mesh=_sc_mesh(),
        compiler_params=_sc_params(),
        scratch_types=[
            pltpu.VMEM((TBL,), jnp.float32),
            pltpu.VMEM((80, 128), jnp.int32),
            pltpu.VMEM((80, 128), jnp.float32),
        ],
    )
    return kern(inv, kd2d)


def _scale_chunk(rows_ref, w_ref, j):
    base = j * 128

    @pl.loop(0, 32)
    def _(q):
        e0 = q * 4
        for u in range(4):
            idx16 = jnp.full((L,), base + e0 + u, jnp.int32)
            w16 = plsc.load_gather(w_ref, [idx16])
            for c in range(8):
                rows_ref[e0 + u, pl.ds(c * L, L)] = (
                    rows_ref[e0 + u, pl.ds(c * L, L)] * w16)


def _scat_body(h_hbm, ks_hbm, dst_hbm, w_hbm, acc_hbm,
               ks_v, dst_v, w_v, rows0_v, rows1_v, acc_sh, sem0, sem1):
    cid = lax.axis_index("c")
    sid = lax.axis_index("s")
    w = sid * NC + cid
    z16 = jnp.zeros((L,), jnp.float32)

    @pl.loop(0, 128)
    def _(i):
        for c in range(8):
            rows0_v[i, pl.ds(c * L, L)] = z16

    for t in range(5):
        pltpu.sync_copy(rows0_v, acc_sh.at[pl.ds(sid * 640 + t * 128, 128)])
    plsc.subcore_barrier()

    for h in range(2):
        pltpu.sync_copy(ks_hbm.at[pl.ds(w * 80 + h * 40, 40)], ks_v)
        pltpu.sync_copy(dst_hbm.at[pl.ds(w * 80 + h * 40, 40)], dst_v)
        pltpu.sync_copy(w_hbm.at[pl.ds(w * EPT + h * 5120, 5120)], w_v)

        pltpu.make_async_copy(h_hbm.at[ks_v.at[0]], rows0_v, sem0).start()

        @pl.loop(0, 20)
        def _(k):
            j0 = k * 2
            j1 = j0 + 1
            pltpu.make_async_copy(h_hbm.at[ks_v.at[j0]], rows0_v, sem0).wait()
            pltpu.make_async_copy(h_hbm.at[ks_v.at[j1]], rows1_v,
                                  sem1).start()
            _scale_chunk(rows0_v, w_v, j0)
            pltpu.sync_copy(rows0_v, acc_sh.at[dst_v.at[j0]], add=True)

            @pl.when(k < 19)
            def _():
                pltpu.make_async_copy(h_hbm.at[ks_v.at[j0 + 2]], rows0_v,
                                      sem0).start()

            pltpu.make_async_copy(h_hbm.at[ks_v.at[j1]], rows1_v, sem1).wait()
            _scale_chunk(rows1_v, w_v, j1)
            pltpu.sync_copy(rows1_v, acc_sh.at[dst_v.at[j1]], add=True)

    plsc.subcore_barrier()
    for t in range(5):
        r0 = sid * 640 + t * 128
        pltpu.sync_copy(acc_sh.at[pl.ds(r0, 128)],
                        acc_hbm.at[pl.ds(cid * NACC + r0, 128)])


def _scatter_layer(h_flat, ks2d, dst2d, w1d):
    kern = pl.kernel(
        _scat_body,
        out_type=jax.ShapeDtypeStruct((NC * NACC, D), jnp.float32),
        mesh=_sc_mesh(),
        compiler_params=_sc_params(),
        scratch_types=[
            pltpu.VMEM((40, 128), jnp.int32),
            pltpu.VMEM((40, 128), jnp.int32),
            pltpu.VMEM((5120,), jnp.float32),
            pltpu.VMEM((128, D), jnp.float32),
            pltpu.VMEM((128, D), jnp.float32),
            pltpu.VMEM_SHARED((NACC, D), jnp.float32),
            pltpu.SemaphoreType.DMA,
            pltpu.SemaphoreType.DMA,
        ],
    )
    return kern(h_flat, ks2d, dst2d, w1d)


def _pool_body(h_hbm, b_hbm, pool_hbm, cnt_hbm, rows_v, bidx_v, cnt_v, acc_sh):
    cid = lax.axis_index("c")
    sid = lax.axis_index("s")
    w = sid * NC + cid
    z16 = jnp.zeros((L,), jnp.float32)
    ones = jnp.ones((L,), jnp.float32)

    @pl.loop(0, PB // L)
    def _(i):
        cnt_v[pl.ds(i * L, L)] = z16

    @pl.loop(0, 128)
    def _(i):
        for c in range(8):
            rows_v[i, pl.ds(c * L, L)] = z16

    @pl.when(sid == 0)
    def _():
        pltpu.sync_copy(rows_v.at[pl.ds(0, PB)], acc_sh)

    plsc.subcore_barrier()

    pltpu.sync_copy(b_hbm.at[w], bidx_v)
    for j in range(3):
        pltpu.sync_copy(h_hbm.at[pl.ds(w * 384 + j * 128, 128)], rows_v)
        pltpu.sync_copy(rows_v, acc_sh.at[bidx_v.at[j]], add=True)
        for k in range(8):
            kb = bidx_v[j, pl.ds(k * L, L)]
            plsc.addupdate_scatter(cnt_v, [kb], ones)

    plsc.subcore_barrier()

    @pl.when(sid == 0)
    def _():
        pltpu.sync_copy(acc_sh, pool_hbm.at[pl.ds(cid * PB, PB)])

    pltpu.sync_copy(cnt_v, cnt_hbm.at[pl.ds(w * PB, PB)])


def _pool(h2p, batch3d):
    kern = pl.kernel(
        _pool_body,
        out_type=(jax.ShapeDtypeStruct((NC * PB, D), jnp.float32),
                  jax.ShapeDtypeStruct((NW * PB,), jnp.float32)),
        mesh=_sc_mesh(),
        compiler_params=_sc_params(),
        scratch_types=[
            pltpu.VMEM((128, D), jnp.float32),
            pltpu.VMEM((3, 128), jnp.int32),
            pltpu.VMEM((PB,), jnp.float32),
            pltpu.VMEM_SHARED((PB, D), jnp.float32),
        ],
    )
    return kern(h2p, batch3d)


def _mm_body(x_ref, w_ref, o_ref):
    o_ref[...] = jnp.dot(x_ref[...], w_ref[...],
                         preferred_element_type=jnp.float32)


def _matmul1(x, wcat):
    return pl.pallas_call(
        _mm_body,
        grid=(10,),
        in_specs=[pl.BlockSpec((1000, D), lambda i: (i, 0)),
                  pl.BlockSpec((D, RK * D), lambda i: (0, 0))],
        out_specs=pl.BlockSpec((1000, RK * D), lambda i: (i, 0)),
        out_shape=jax.ShapeDtypeStruct((N, RK * D), jnp.float32),
    )(x, wcat)


def _comb_mm_body(a0_ref, a1_ref, hr_ref, b_ref, w_ref, o_ref):
    h = a0_ref[...] + a1_ref[...] + hr_ref[...] + b_ref[...]
    h = jnp.maximum(h, 0.0)
    o_ref[...] = jnp.dot(h, w_ref[...], preferred_element_type=jnp.float32)


def _comb_matmul(acc0, acc1, hroot, b, wcat):
    return pl.pallas_call(
        _comb_mm_body,
        grid=(10,),
        in_specs=[pl.BlockSpec((1000, D), lambda i: (i, 0)),
                  pl.BlockSpec((1000, D), lambda i: (i, 0)),
                  pl.BlockSpec((1000, D), lambda i: (i, 0)),
                  pl.BlockSpec((1, D), lambda i: (0, 0)),
                  pl.BlockSpec((D, RK * D), lambda i: (0, 0))],
        out_specs=pl.BlockSpec((1000, RK * D), lambda i: (i, 0)),
        out_shape=jax.ShapeDtypeStruct((N, RK * D), jnp.float32),
    )(acc0, acc1, hroot, b, wcat)


def _comb_body(a0_ref, a1_ref, hr_ref, b_ref, o_ref):
    h = a0_ref[...] + a1_ref[...] + hr_ref[...] + b_ref[...]
    o_ref[...] = jnp.maximum(h, 0.0)


def _comb_relu(acc0, acc1, hroot, b):
    return pl.pallas_call(
        _comb_body,
        grid=(10,),
        in_specs=[pl.BlockSpec((1000, D), lambda i: (i, 0)),
                  pl.BlockSpec((1000, D), lambda i: (i, 0)),
                  pl.BlockSpec((1000, D), lambda i: (i, 0)),
                  pl.BlockSpec((1, D), lambda i: (0, 0))],
        out_specs=pl.BlockSpec((1000, D), lambda i: (i, 0)),
        out_shape=jax.ShapeDtypeStruct((NPOOL, D), jnp.float32),
    )(acc0, acc1, hroot, b)


def _inv_body(c_ref, o_ref):
    s = jnp.sum(c_ref[...], axis=0)
    gid = (128 * lax.broadcasted_iota(jnp.int32, (TBL // 128, 128), 0)
           + lax.broadcasted_iota(jnp.int32, (TBL // 128, 128), 1))
    o_ref[...] = jnp.where(gid < PAD_BIN, 1.0 / jnp.maximum(s, 1.0), 0.0)


def _inv_counts(cnt32):
    c3 = cnt32.reshape(NW, TBL // 128, 128)
    out = pl.pallas_call(
        _inv_body,
        out_shape=jax.ShapeDtypeStruct((TBL // 128, 128), jnp.float32),
    )(c3)
    return out.reshape(TBL)


def _mlp_body(p_ref, c_ref, w1_ref, b1_ref, w2_ref, b2_ref, o_ref):
    cnt = jnp.sum(c_ref[...], axis=0)[:G]
    g = p_ref[0:G, :] + p_ref[PB:PB + G, :]
    g = g / jnp.maximum(cnt, 1.0)[:, None]
    a = jnp.maximum(jnp.dot(g, w1_ref[...],
                            preferred_element_type=jnp.float32)
                    + b1_ref[...], 0.0)
    logits = jnp.dot(a, w2_ref[...],
                     preferred_element_type=jnp.float32) + b2_ref[...]
    m = jnp.max(logits, axis=1, keepdims=True)
    s = logits - m
    o_ref[...] = s - jnp.log(jnp.sum(jnp.exp(s), axis=1, keepdims=True))


def _mlp_head(pool2, cntb, wm1, bm1, wm2, bm2):
    return pl.pallas_call(
        _mlp_body,
        in_specs=[pl.BlockSpec((NC * PB, D), lambda: (0, 0)),
                  pl.BlockSpec((NW, PB), lambda: (0, 0)),
                  pl.BlockSpec((D, MLP), lambda: (0, 0)),
                  pl.BlockSpec((1, MLP), lambda: (0, 0)),
                  pl.BlockSpec((MLP, NCLS), lambda: (0, 0)),
                  pl.BlockSpec((1, NCLS), lambda: (0, 0))],
        out_specs=pl.BlockSpec((G, NCLS), lambda: (0, 0)),
        out_shape=jax.ShapeDtypeStruct((G, NCLS), jnp.float32),
    )(pool2, cntb, wm1, bm1, wm2, bm2)


def kernel(x, edge_index, edge_type, batch, W1, root1, b1, W2, root2, b2,
           Wm1, bm1, Wm2, bm2):
    f32, i32 = jnp.float32, jnp.int32
    src = edge_index[0]
    dst = edge_index[1]
    et = edge_type

    pad = EP - E
    ks2d = jnp.concatenate(
        [src * RK + et, jnp.zeros((pad,), i32)]).reshape(EROWS, 128)
    kd2d = jnp.concatenate(
        [dst * NREL + et, jnp.full((pad,), PAD_BIN, i32)]).reshape(EROWS, 128)
    dst2d = jnp.concatenate(
        [dst, jnp.zeros((pad,), i32)]).reshape(EROWS, 128)
    batch3d = jnp.concatenate(
        [batch, jnp.full((NPOOL - N,), G, i32)]).reshape(NW, 3, 128)

    wcat1 = jnp.concatenate([W1.transpose(1, 0, 2).reshape(D, NREL * D),
                             root1], axis=1)
    wcat2 = jnp.concatenate([W2.transpose(1, 0, 2).reshape(D, NREL * D),
                             root2], axis=1)

    cnt32 = _counts(kd2d).reshape(NW, TBL)
    inv = _inv_counts(cnt32)
    w1d = _scales(inv, kd2d).reshape(EP)

    h1cat = _matmul1(x, wcat1)
    acc1 = _scatter_layer(h1cat.reshape(RK * N, D), ks2d, dst2d, w1d)
    acc1 = acc1.reshape(NC, NACC, D)

    h2cat = _comb_matmul(acc1[0, :N], acc1[1, :N],
                         h1cat[:, NREL * D:], b1.reshape(1, D), wcat2)
    acc2 = _scatter_layer(h2cat.reshape(RK * N, D), ks2d, dst2d, w1d)
    acc2 = acc2.reshape(NC, NACC, D)

    h2p = _comb_relu(acc2[0, :N], acc2[1, :N],
                     h2cat[:, NREL * D:], b2.reshape(1, D))

    pool2, cntb = _pool(h2p, batch3d)
    cntb = cntb.reshape(NW, PB)
    return _mlp_head(pool2, cntb, Wm1, bm1.reshape(1, MLP),
                     Wm2, bm2.reshape(1, NCLS))


# speedup vs baseline: 16.1877x; 1.3202x over previous
import dataclasses
import functools

import jax
import jax.numpy as jnp
from jax import lax
from jax.experimental import pallas as pl
from jax.experimental.pallas import tpu as pltpu
from jax.experimental.pallas import tpu_sc as plsc

N = 10000
E = 320000
NREL = 8
G = 64
D = 128
RK = NREL + 1
MLP = 256
NCLS = 10

NC = 2
NS = 16
NW = NC * NS
L = 16

EPT = 10240
EP = NW * EPT
EROWS = EP // 128
TBL = 80128
PAD_BIN = 80000
NACC = 10240
NPOOL = 12288
PB = 80
SEG = 40
NSEG = 160 // SEG
C0SEG = 3


def _sc_mesh():
    return plsc.VectorSubcoreMesh(core_axis_name="c", subcore_axis_name="s")


def _sc_params():
    cp = pltpu.CompilerParams()
    if "needs_layout_passes" in pltpu.CompilerParams.__dataclass_fields__:
        cp = dataclasses.replace(cp, needs_layout_passes=False)
    return cp


def _wid():
    return lax.axis_index("s") * NC + lax.axis_index("c")


def _count_body(kd_hbm, cnt_hbm, tbl_v, kd_v):
    w = _wid()
    z16 = jnp.zeros((L,), jnp.float32)
    ones = jnp.ones((L,), jnp.float32)

    @pl.loop(0, TBL // (4 * L))
    def _(i):
        for u in range(4):
            tbl_v[pl.ds(i * 4 * L + u * L, L)] = z16

    pltpu.sync_copy(kd_hbm.at[pl.ds(w * 80, 80)], kd_v)

    @pl.loop(0, 80)
    def _(j):
        for k in range(8):
            kk = kd_v[j, pl.ds(k * L, L)]
            plsc.addupdate_scatter(tbl_v, [kk], ones)

    pltpu.sync_copy(tbl_v, cnt_hbm.at[pl.ds(w * TBL, TBL)])


def _counts(kd2d):
    kern = pl.kernel(
        _count_body,
        out_type=jax.ShapeDtypeStruct((NW * TBL,), jnp.float32),
        mesh=_sc_mesh(),
        compiler_params=_sc_params(),
        scratch_types=[
            pltpu.VMEM((TBL,), jnp.float32),
            pltpu.VMEM((80, 128), jnp.int32),
        ],
    )
    return kern(kd2d)


def _scale_body(inv_hbm, kd_hbm, w_hbm, inv_v, kd_v, wout_v):
    w = _wid()
    pltpu.sync_copy(inv_hbm, inv_v)
    pltpu.sync_copy(kd_hbm.at[pl.ds(w * 80, 80)], kd_v)

    @pl.loop(0, 80)
    def _(j):
        for k in range(8):
            kk = kd_v[j, pl.ds(k * L, L)]
            wout_v[j, pl.ds(k * L, L)] = plsc.load_gather(inv_v, [kk])

    pltpu.sync_copy(wout_v, w_hbm.at[pl.ds(w * 80, 80)])


def _scales(inv, kd2d):
    kern = pl.kernel(
        _scale_body,
        out_type=jax.ShapeDtypeStruct((EROWS, 128), jnp.float32),
        mesh=_sc_mesh(),
        compiler_params=_sc_params(),
        scratch_types=[
            pltpu.VMEM((TBL,), jnp.float32),
            pltpu.VMEM((80, 128), jnp.int32),
            pltpu.VMEM((80, 128), jnp.float32),
        ],
    )
    return kern(inv, kd2d)


def _scale_chunk(rows_ref, w_ref, j):
    base = j * 128

    @pl.loop(0, 16)
    def _(q):
        e0 = q * 8
        for u in range(8):
            idx16 = jnp.full((L,), base + e0 + u, jnp.int32)
            w16 = plsc.load_gather(w_ref, [idx16])
            for c in range(8):
                rows_ref[e0 + u, pl.ds(c * L, L)] = (
                    rows_ref[e0 + u, pl.ds(c * L, L)] * w16)


def _scat_body(h_hbm, ks_hbm, dst_hbm, w_hbm, acc_hbm,
               ks_v, dst_v, w_v, rows0_v, rows1_v, acc_sh, sem0, sem1):
    cid = lax.axis_index("c")
    sid = lax.axis_index("s")
    w = sid * NC + cid
    z16 = jnp.zeros((L,), jnp.float32)

    @pl.loop(0, 128)
    def _(i):
        for c in range(8):
            rows0_v[i, pl.ds(c * L, L)] = z16

    for t in range(5):
        pltpu.sync_copy(rows0_v, acc_sh.at[pl.ds(sid * 640 + t * 128, 128)])
    plsc.subcore_barrier()

    nseg = jnp.where(cid == 0, C0SEG, NSEG - C0SEG)
    base_row = sid * 160 + cid * (C0SEG * SEG)

    @pl.loop(0, max(C0SEG, NSEG - C0SEG))
    def _(sgi):
        @pl.when(sgi < nseg)
        def _():
            r0 = base_row + sgi * SEG
            pltpu.sync_copy(ks_hbm.at[pl.ds(r0, SEG)], ks_v)
            pltpu.sync_copy(dst_hbm.at[pl.ds(r0, SEG)], dst_v)
            pltpu.sync_copy(w_hbm.at[pl.ds(r0 * 128, SEG * 128)], w_v)

            pltpu.make_async_copy(h_hbm.at[ks_v.at[0]], rows0_v,
                                  sem0).start()

            @pl.loop(0, SEG // 2)
            def _(k):
                j0 = k * 2
                j1 = j0 + 1
                pltpu.make_async_copy(h_hbm.at[ks_v.at[j0]], rows0_v,
                                      sem0).wait()
                pltpu.make_async_copy(h_hbm.at[ks_v.at[j1]], rows1_v,
                                      sem1).start()
                _scale_chunk(rows0_v, w_v, j0)
                pltpu.sync_copy(rows0_v, acc_sh.at[dst_v.at[j0]], add=True)

                @pl.when(k < SEG // 2 - 1)
                def _():
                    pltpu.make_async_copy(h_hbm.at[ks_v.at[j0 + 2]], rows0_v,
                                          sem0).start()

                pltpu.make_async_copy(h_hbm.at[ks_v.at[j1]], rows1_v,
                                      sem1).wait()
                _scale_chunk(rows1_v, w_v, j1)
                pltpu.sync_copy(rows1_v, acc_sh.at[dst_v.at[j1]], add=True)

    plsc.subcore_barrier()
    for t in range(5):
        r0 = sid * 640 + t * 128
        pltpu.sync_copy(acc_sh.at[pl.ds(r0, 128)],
                        acc_hbm.at[pl.ds(cid * NACC + r0, 128)])


def _scatter_layer(h_flat, ks2d, dst2d, w1d):
    kern = pl.kernel(
        _scat_body,
        out_type=jax.ShapeDtypeStruct((NC * NACC, D), jnp.float32),
        mesh=_sc_mesh(),
        compiler_params=_sc_params(),
        scratch_types=[
            pltpu.VMEM((SEG, 128), jnp.int32),
            pltpu.VMEM((SEG, 128), jnp.int32),
            pltpu.VMEM((SEG * 128,), jnp.float32),
            pltpu.VMEM((128, D), jnp.float32),
            pltpu.VMEM((128, D), jnp.float32),
            pltpu.VMEM_SHARED((NACC, D), jnp.float32),
            pltpu.SemaphoreType.DMA,
            pltpu.SemaphoreType.DMA,
        ],
    )
    return kern(h_flat, ks2d, dst2d, w1d)


def _pool_body(h_hbm, b_hbm, pool_hbm, cnt_hbm, rows_v, bidx_v, cnt_v, acc_sh):
    cid = lax.axis_index("c")
    sid = lax.axis_index("s")
    w = sid * NC + cid
    z16 = jnp.zeros((L,), jnp.float32)
    ones = jnp.ones((L,), jnp.float32)

    @pl.loop(0, PB // L)
    def _(i):
        cnt_v[pl.ds(i * L, L)] = z16

    @pl.loop(0, 128)
    def _(i):
        for c in range(8):
            rows_v[i, pl.ds(c * L, L)] = z16

    @pl.when(sid == 0)
    def _():
        pltpu.sync_copy(rows_v.at[pl.ds(0, PB)], acc_sh)

    plsc.subcore_barrier()

    pltpu.sync_copy(b_hbm.at[w], bidx_v)
    for j in range(3):
        pltpu.sync_copy(h_hbm.at[pl.ds(w * 384 + j * 128, 128)], rows_v)
        pltpu.sync_copy(rows_v, acc_sh.at[bidx_v.at[j]], add=True)
        for k in range(8):
            kb = bidx_v[j, pl.ds(k * L, L)]
            plsc.addupdate_scatter(cnt_v, [kb], ones)

    plsc.subcore_barrier()

    @pl.when(sid == 0)
    def _():
        pltpu.sync_copy(acc_sh, pool_hbm.at[pl.ds(cid * PB, PB)])

    pltpu.sync_copy(cnt_v, cnt_hbm.at[pl.ds(w * PB, PB)])


def _pool(h2p, batch3d):
    kern = pl.kernel(
        _pool_body,
        out_type=(jax.ShapeDtypeStruct((NC * PB, D), jnp.float32),
                  jax.ShapeDtypeStruct((NW * PB,), jnp.float32)),
        mesh=_sc_mesh(),
        compiler_params=_sc_params(),
        scratch_types=[
            pltpu.VMEM((128, D), jnp.float32),
            pltpu.VMEM((3, 128), jnp.int32),
            pltpu.VMEM((PB,), jnp.float32),
            pltpu.VMEM_SHARED((PB, D), jnp.float32),
        ],
    )
    return kern(h2p, batch3d)


def _mm_body(x_ref, w_ref, o_ref):
    o_ref[...] = jnp.dot(x_ref[...], w_ref[...],
                         preferred_element_type=jnp.float32)


def _matmul1(x, wcat):
    return pl.pallas_call(
        _mm_body,
        grid=(10,),
        in_specs=[pl.BlockSpec((1000, D), lambda i: (i, 0)),
                  pl.BlockSpec((D, RK * D), lambda i: (0, 0))],
        out_specs=pl.BlockSpec((1000, RK * D), lambda i: (i, 0)),
        out_shape=jax.ShapeDtypeStruct((N, RK * D), jnp.float32),
    )(x, wcat)


def _comb_mm_body(a0_ref, a1_ref, hr_ref, b_ref, w_ref, o_ref):
    h = a0_ref[...] + a1_ref[...] + hr_ref[...] + b_ref[...]
    h = jnp.maximum(h, 0.0)
    o_ref[...] = jnp.dot(h, w_ref[...], preferred_element_type=jnp.float32)


def _comb_matmul(acc0, acc1, hroot, b, wcat):
    return pl.pallas_call(
        _comb_mm_body,
        grid=(10,),
        in_specs=[pl.BlockSpec((1000, D), lambda i: (i, 0)),
                  pl.BlockSpec((1000, D), lambda i: (i, 0)),
                  pl.BlockSpec((1000, D), lambda i: (i, 0)),
                  pl.BlockSpec((1, D), lambda i: (0, 0)),
                  pl.BlockSpec((D, RK * D), lambda i: (0, 0))],
        out_specs=pl.BlockSpec((1000, RK * D), lambda i: (i, 0)),
        out_shape=jax.ShapeDtypeStruct((N, RK * D), jnp.float32),
    )(acc0, acc1, hroot, b, wcat)


def _comb_body(a0_ref, a1_ref, hr_ref, b_ref, o_ref):
    h = a0_ref[...] + a1_ref[...] + hr_ref[...] + b_ref[...]
    o_ref[...] = jnp.maximum(h, 0.0)


def _comb_relu(acc0, acc1, hroot, b):
    return pl.pallas_call(
        _comb_body,
        grid=(10,),
        in_specs=[pl.BlockSpec((1000, D), lambda i: (i, 0)),
                  pl.BlockSpec((1000, D), lambda i: (i, 0)),
                  pl.BlockSpec((1000, D), lambda i: (i, 0)),
                  pl.BlockSpec((1, D), lambda i: (0, 0))],
        out_specs=pl.BlockSpec((1000, D), lambda i: (i, 0)),
        out_shape=jax.ShapeDtypeStruct((NPOOL, D), jnp.float32),
    )(acc0, acc1, hroot, b)


def _inv_body(c_ref, o_ref):
    s = jnp.sum(c_ref[...], axis=0)
    gid = (128 * lax.broadcasted_iota(jnp.int32, (TBL // 128, 128), 0)
           + lax.broadcasted_iota(jnp.int32, (TBL // 128, 128), 1))
    o_ref[...] = jnp.where(gid < PAD_BIN, 1.0 / jnp.maximum(s, 1.0), 0.0)


def _inv_counts(cnt32):
    c3 = cnt32.reshape(NW, TBL // 128, 128)
    out = pl.pallas_call(
        _inv_body,
        out_shape=jax.ShapeDtypeStruct((TBL // 128, 128), jnp.float32),
    )(c3)
    return out.reshape(TBL)


def _mlp_body(p_ref, c_ref, w1_ref, b1_ref, w2_ref, b2_ref, o_ref):
    cnt = jnp.sum(c_ref[...], axis=0)[:G]
    g = p_ref[0:G, :] + p_ref[PB:PB + G, :]
    g = g / jnp.maximum(cnt, 1.0)[:, None]
    a = jnp.maximum(jnp.dot(g, w1_ref[...],
                            preferred_element_type=jnp.float32)
                    + b1_ref[...], 0.0)
    logits = jnp.dot(a, w2_ref[...],
                     preferred_element_type=jnp.float32) + b2_ref[...]
    m = jnp.max(logits, axis=1, keepdims=True)
    s = logits - m
    o_ref[...] = s - jnp.log(jnp.sum(jnp.exp(s), axis=1, keepdims=True))


def _mlp_head(pool2, cntb, wm1, bm1, wm2, bm2):
    return pl.pallas_call(
        _mlp_body,
        in_specs=[pl.BlockSpec((NC * PB, D), lambda: (0, 0)),
                  pl.BlockSpec((NW, PB), lambda: (0, 0)),
                  pl.BlockSpec((D, MLP), lambda: (0, 0)),
                  pl.BlockSpec((1, MLP), lambda: (0, 0)),
                  pl.BlockSpec((MLP, NCLS), lambda: (0, 0)),
                  pl.BlockSpec((1, NCLS), lambda: (0, 0))],
        out_specs=pl.BlockSpec((G, NCLS), lambda: (0, 0)),
        out_shape=jax.ShapeDtypeStruct((G, NCLS), jnp.float32),
    )(pool2, cntb, wm1, bm1, wm2, bm2)


def kernel(x, edge_index, edge_type, batch, W1, root1, b1, W2, root2, b2,
           Wm1, bm1, Wm2, bm2):
    f32, i32 = jnp.float32, jnp.int32
    src = edge_index[0]
    dst = edge_index[1]
    et = edge_type

    pad = EP - E
    ks2d = jnp.concatenate(
        [src * RK + et, jnp.zeros((pad,), i32)]).reshape(EROWS, 128)
    kd2d = jnp.concatenate(
        [dst * NREL + et, jnp.full((pad,), PAD_BIN, i32)]).reshape(EROWS, 128)
    dst2d = jnp.concatenate(
        [dst, jnp.zeros((pad,), i32)]).reshape(EROWS, 128)
    batch3d = jnp.concatenate(
        [batch, jnp.full((NPOOL - N,), G, i32)]).reshape(NW, 3, 128)

    wcat1 = jnp.concatenate([W1.transpose(1, 0, 2).reshape(D, NREL * D),
                             root1], axis=1)
    wcat2 = jnp.concatenate([W2.transpose(1, 0, 2).reshape(D, NREL * D),
                             root2], axis=1)

    h1cat = _matmul1(x, wcat1)

    cnt32 = _counts(kd2d).reshape(NW, TBL)
    inv = _inv_counts(cnt32)
    w1d = _scales(inv, kd2d).reshape(EP)

    acc1 = _scatter_layer(h1cat.reshape(RK * N, D), ks2d, dst2d, w1d)
    acc1 = acc1.reshape(NC, NACC, D)

    h2cat = _comb_matmul(acc1[0, :N], acc1[1, :N],
                         h1cat[:, NREL * D:], b1.reshape(1, D), wcat2)
    acc2 = _scatter_layer(h2cat.reshape(RK * N, D), ks2d, dst2d, w1d)
    acc2 = acc2.reshape(NC, NACC, D)

    h2p = _comb_relu(acc2[0, :N], acc2[1, :N],
                     h2cat[:, NREL * D:], b2.reshape(1, D))

    pool2, cntb = _pool(h2p, batch3d)
    cntb = cntb.reshape(NW, PB)
    return _mlp_head(pool2, cntb, Wm1, bm1.reshape(1, MLP),
                     Wm2, bm2.reshape(1, NCLS))


# speedup vs baseline: 16.1887x; 1.0001x over previous
import dataclasses

import jax
import jax.numpy as jnp
from jax import lax
from jax.experimental import pallas as pl
from jax.experimental.pallas import tpu as pltpu
from jax.experimental.pallas import tpu_sc as plsc

N = 10000
E = 320000
NREL = 8
G = 64
D = 128
RK = NREL + 1
MLP = 256
NCLS = 10

NC = 2
NS = 16
NW = NC * NS
L = 16

EPT = 10240
EP = NW * EPT
EROWS = EP // 128
TBL = 80128
PAD_BIN = 80000
NACC = 10240
NPOOL = 12288
PB = 80
SEG = 40
NSEG = 160 // SEG
C0SEG = 3


def _sc_mesh():
    return plsc.VectorSubcoreMesh(core_axis_name="c", subcore_axis_name="s")


def _sc_params():
    cp = pltpu.CompilerParams()
    if "needs_layout_passes" in pltpu.CompilerParams.__dataclass_fields__:
        cp = dataclasses.replace(cp, needs_layout_passes=False)
    return cp


def _wid():
    return lax.axis_index("s") * NC + lax.axis_index("c")


def _count_body(kd_hbm, cnt_hbm, tbl_v, kd_v):
    w = _wid()
    z16 = jnp.zeros((L,), jnp.float32)
    ones = jnp.ones((L,), jnp.float32)

    @pl.loop(0, TBL // (4 * L))
    def _(i):
        for u in range(4):
            tbl_v[pl.ds(i * 4 * L + u * L, L)] = z16

    pltpu.sync_copy(kd_hbm.at[pl.ds(w * 80, 80)], kd_v)

    @pl.loop(0, 80)
    def _(j):
        for k in range(8):
            kk = kd_v[j, pl.ds(k * L, L)]
            plsc.addupdate_scatter(tbl_v, [kk], ones)

    pltpu.sync_copy(tbl_v, cnt_hbm.at[pl.ds(w * TBL, TBL)])


def _counts(kd2d):
    kern = pl.kernel(
        _count_body,
        out_type=jax.ShapeDtypeStruct((NW * TBL,), jnp.float32),
        mesh=_sc_mesh(),
        compiler_params=_sc_params(),
        scratch_types=[
            pltpu.VMEM((TBL,), jnp.float32),
            pltpu.VMEM((80, 128), jnp.int32),
        ],
    )
    return kern(kd2d)


def _scale_body(inv_hbm, kd_hbm, w_hbm, inv_v, kd_v, wout_v):
    w = _wid()
    pltpu.sync_copy(inv_hbm, inv_v)
    pltpu.sync_copy(kd_hbm.at[pl.ds(w * 80, 80)], kd_v)

    @pl.loop(0, 80)
    def _(j):
        for k in range(8):
            kk = kd_v[j, pl.ds(k * L, L)]
            wout_v[j, pl.ds(k * L, L)] = plsc.load_gather(inv_v, [kk])

    pltpu.sync_copy(wout_v, w_hbm.at[pl.ds(w * 80, 80)])


def _scales(inv, kd2d):
    kern = pl.kernel(
        _scale_body,
        out_type=jax.ShapeDtypeStruct((EROWS, 128), jnp.float32),
        mesh=_sc_mesh(),
        compiler_params=_sc_params(),
        scratch_types=[
            pltpu.VMEM((TBL,), jnp.float32),
            pltpu.VMEM((80, 128), jnp.int32),
            pltpu.VMEM((80, 128), jnp.float32),
        ],
    )
    return kern(inv, kd2d)


def _scale_chunk(rows_ref, w_ref, j):
    base = j * 128

    @pl.loop(0, 16)
    def _(q):
        e0 = q * 8
        for u in range(8):
            idx16 = jnp.full((L,), base + e0 + u, jnp.int32)
            w16 = plsc.load_gather(w_ref, [idx16])
            for c in range(8):
                rows_ref[e0 + u, pl.ds(c * L, L)] = (
                    rows_ref[e0 + u, pl.ds(c * L, L)] * w16)


def _scat_body(h_hbm, ks_hbm, dst_hbm, w_hbm, acc_hbm,
               ks_v, dst_v, w_v, rows0_v, rows1_v, acc_sh, sem0, sem1):
    cid = lax.axis_index("c")
    sid = lax.axis_index("s")
    w = sid * NC + cid
    z16 = jnp.zeros((L,), jnp.float32)

    @pl.loop(0, 128)
    def _(i):
        for c in range(8):
            rows0_v[i, pl.ds(c * L, L)] = z16

    for t in range(5):
        pltpu.sync_copy(rows0_v, acc_sh.at[pl.ds(sid * 640 + t * 128, 128)])
    plsc.subcore_barrier()

    nseg = jnp.where(cid == 0, C0SEG, NSEG - C0SEG)
    base_row = sid * 160 + cid * (C0SEG * SEG)

    @pl.loop(0, max(C0SEG, NSEG - C0SEG))
    def _(sgi):
        @pl.when(sgi < nseg)
        def _():
            r0 = base_row + sgi * SEG
            pltpu.sync_copy(ks_hbm.at[pl.ds(r0, SEG)], ks_v)
            pltpu.sync_copy(dst_hbm.at[pl.ds(r0, SEG)], dst_v)
            pltpu.sync_copy(w_hbm.at[pl.ds(r0 * 128, SEG * 128)], w_v)

            pltpu.make_async_copy(h_hbm.at[ks_v.at[0]], rows0_v,
                                  sem0).start()

            @pl.loop(0, SEG // 2)
            def _(k):
                j0 = k * 2
                j1 = j0 + 1
                pltpu.make_async_copy(h_hbm.at[ks_v.at[j0]], rows0_v,
                                      sem0).wait()
                pltpu.make_async_copy(h_hbm.at[ks_v.at[j1]], rows1_v,
                                      sem1).start()
                _scale_chunk(rows0_v, w_v, j0)
                pltpu.sync_copy(rows0_v, acc_sh.at[dst_v.at[j0]], add=True)

                @pl.when(k < SEG // 2 - 1)
                def _():
                    pltpu.make_async_copy(h_hbm.at[ks_v.at[j0 + 2]], rows0_v,
                                          sem0).start()

                pltpu.make_async_copy(h_hbm.at[ks_v.at[j1]], rows1_v,
                                      sem1).wait()
                _scale_chunk(rows1_v, w_v, j1)
                pltpu.sync_copy(rows1_v, acc_sh.at[dst_v.at[j1]], add=True)

    plsc.subcore_barrier()
    for t in range(5):
        r0 = sid * 640 + t * 128
        pltpu.sync_copy(acc_sh.at[pl.ds(r0, 128)],
                        acc_hbm.at[pl.ds(cid * NACC + r0, 128)])


def _scatter_layer(h_flat, ks2d, dst2d, w1d):
    kern = pl.kernel(
        _scat_body,
        out_type=jax.ShapeDtypeStruct((NC * NACC, D), jnp.float32),
        mesh=_sc_mesh(),
        compiler_params=_sc_params(),
        scratch_types=[
            pltpu.VMEM((SEG, 128), jnp.int32),
            pltpu.VMEM((SEG, 128), jnp.int32),
            pltpu.VMEM((SEG * 128,), jnp.float32),
            pltpu.VMEM((128, D), jnp.float32),
            pltpu.VMEM((128, D), jnp.float32),
            pltpu.VMEM_SHARED((NACC, D), jnp.float32),
            pltpu.SemaphoreType.DMA,
            pltpu.SemaphoreType.DMA,
        ],
    )
    return kern(h_flat, ks2d, dst2d, w1d)


def _pool_body(h_hbm, b_hbm, pool_hbm, cnt_hbm, rows_v, bidx_v, cnt_v, acc_sh):
    cid = lax.axis_index("c")
    sid = lax.axis_index("s")
    w = sid * NC + cid
    z16 = jnp.zeros((L,), jnp.float32)
    ones = jnp.ones((L,), jnp.float32)

    @pl.loop(0, PB // L)
    def _(i):
        cnt_v[pl.ds(i * L, L)] = z16

    @pl.loop(0, 128)
    def _(i):
        for c in range(8):
            rows_v[i, pl.ds(c * L, L)] = z16

    @pl.when(sid == 0)
    def _():
        pltpu.sync_copy(rows_v.at[pl.ds(0, PB)], acc_sh)

    plsc.subcore_barrier()

    pltpu.sync_copy(b_hbm.at[w], bidx_v)
    for j in range(3):
        pltpu.sync_copy(h_hbm.at[pl.ds(w * 384 + j * 128, 128)], rows_v)
        pltpu.sync_copy(rows_v, acc_sh.at[bidx_v.at[j]], add=True)
        for k in range(8):
            kb = bidx_v[j, pl.ds(k * L, L)]
            plsc.addupdate_scatter(cnt_v, [kb], ones)

    plsc.subcore_barrier()

    @pl.when(sid == 0)
    def _():
        pltpu.sync_copy(acc_sh, pool_hbm.at[pl.ds(cid * PB, PB)])

    pltpu.sync_copy(cnt_v, cnt_hbm.at[pl.ds(w * PB, PB)])


def _pool(h2p, batch3d):
    kern = pl.kernel(
        _pool_body,
        out_type=(jax.ShapeDtypeStruct((NC * PB, D), jnp.float32),
                  jax.ShapeDtypeStruct((NW * PB,), jnp.float32)),
        mesh=_sc_mesh(),
        compiler_params=_sc_params(),
        scratch_types=[
            pltpu.VMEM((128, D), jnp.float32),
            pltpu.VMEM((3, 128), jnp.int32),
            pltpu.VMEM((PB,), jnp.float32),
            pltpu.VMEM_SHARED((PB, D), jnp.float32),
        ],
    )
    return kern(h2p, batch3d)


def _mm_body(x_ref, w_ref, o_ref):
    o_ref[...] = jnp.dot(x_ref[...], w_ref[...],
                         preferred_element_type=jnp.float32)


def _matmul1(x, wcat):
    return pl.pallas_call(
        _mm_body,
        grid=(10,),
        in_specs=[pl.BlockSpec((1000, D), lambda i: (i, 0)),
                  pl.BlockSpec((D, RK * D), lambda i: (0, 0))],
        out_specs=pl.BlockSpec((1000, RK * D), lambda i: (i, 0)),
        out_shape=jax.ShapeDtypeStruct((N, RK * D), jnp.float32),
    )(x, wcat)


def _comb_mm_body(a0_ref, a1_ref, hr_ref, b_ref, w_ref, o_ref):
    h = a0_ref[...] + a1_ref[...] + hr_ref[...] + b_ref[...]
    h = jnp.maximum(h, 0.0)
    o_ref[...] = jnp.dot(h, w_ref[...], preferred_element_type=jnp.float32)


def _comb_matmul(acc0, acc1, hroot, b, wcat):
    return pl.pallas_call(
        _comb_mm_body,
        grid=(10,),
        in_specs=[pl.BlockSpec((1000, D), lambda i: (i, 0)),
                  pl.BlockSpec((1000, D), lambda i: (i, 0)),
                  pl.BlockSpec((1000, D), lambda i: (i, 0)),
                  pl.BlockSpec((1, D), lambda i: (0, 0)),
                  pl.BlockSpec((D, RK * D), lambda i: (0, 0))],
        out_specs=pl.BlockSpec((1000, RK * D), lambda i: (i, 0)),
        out_shape=jax.ShapeDtypeStruct((N, RK * D), jnp.float32),
    )(acc0, acc1, hroot, b, wcat)


def _comb_body(a0_ref, a1_ref, hr_ref, b_ref, o_ref):
    h = a0_ref[...] + a1_ref[...] + hr_ref[...] + b_ref[...]
    o_ref[...] = jnp.maximum(h, 0.0)


def _comb_relu(acc0, acc1, hroot, b):
    return pl.pallas_call(
        _comb_body,
        grid=(10,),
        in_specs=[pl.BlockSpec((1000, D), lambda i: (i, 0)),
                  pl.BlockSpec((1000, D), lambda i: (i, 0)),
                  pl.BlockSpec((1000, D), lambda i: (i, 0)),
                  pl.BlockSpec((1, D), lambda i: (0, 0))],
        out_specs=pl.BlockSpec((1000, D), lambda i: (i, 0)),
        out_shape=jax.ShapeDtypeStruct((NPOOL, D), jnp.float32),
    )(acc0, acc1, hroot, b)


def _inv_body(c_ref, o_ref):
    s = jnp.sum(c_ref[...], axis=0)
    gid = (128 * lax.broadcasted_iota(jnp.int32, (TBL // 128, 128), 0)
           + lax.broadcasted_iota(jnp.int32, (TBL // 128, 128), 1))
    o_ref[...] = jnp.where(gid < PAD_BIN, 1.0 / jnp.maximum(s, 1.0), 0.0)


def _inv_counts(cnt32):
    c3 = cnt32.reshape(NW, TBL // 128, 128)
    out = pl.pallas_call(
        _inv_body,
        out_shape=jax.ShapeDtypeStruct((TBL // 128, 128), jnp.float32),
    )(c3)
    return out.reshape(TBL)


def _mlp_body(p_ref, c_ref, w1_ref, b1_ref, w2_ref, b2_ref, o_ref):
    cnt = jnp.sum(c_ref[...], axis=0)[:G]
    g = p_ref[0:G, :] + p_ref[PB:PB + G, :]
    g = g / jnp.maximum(cnt, 1.0)[:, None]
    a = jnp.maximum(jnp.dot(g, w1_ref[...],
                            preferred_element_type=jnp.float32)
                    + b1_ref[...], 0.0)
    logits = jnp.dot(a, w2_ref[...],
                     preferred_element_type=jnp.float32) + b2_ref[...]
    m = jnp.max(logits, axis=1, keepdims=True)
    s = logits - m
    o_ref[...] = s - jnp.log(jnp.sum(jnp.exp(s), axis=1, keepdims=True))


def _mlp_head(pool2, cntb, wm1, bm1, wm2, bm2):
    return pl.pallas_call(
        _mlp_body,
        in_specs=[pl.BlockSpec((NC * PB, D), lambda: (0, 0)),
                  pl.BlockSpec((NW, PB), lambda: (0, 0)),
                  pl.BlockSpec((D, MLP), lambda: (0, 0)),
                  pl.BlockSpec((1, MLP), lambda: (0, 0)),
                  pl.BlockSpec((MLP, NCLS), lambda: (0, 0)),
                  pl.BlockSpec((1, NCLS), lambda: (0, 0))],
        out_specs=pl.BlockSpec((G, NCLS), lambda: (0, 0)),
        out_shape=jax.ShapeDtypeStruct((G, NCLS), jnp.float32),
    )(pool2, cntb, wm1, bm1, wm2, bm2)


def kernel(x, edge_index, edge_type, batch, W1, root1, b1, W2, root2, b2,
           Wm1, bm1, Wm2, bm2):
    f32, i32 = jnp.float32, jnp.int32
    src = edge_index[0]
    dst = edge_index[1]
    et = edge_type

    pad = EP - E
    ks2d = jnp.concatenate(
        [src * RK + et, jnp.zeros((pad,), i32)]).reshape(EROWS, 128)
    kd2d = jnp.concatenate(
        [dst * NREL + et, jnp.full((pad,), PAD_BIN, i32)]).reshape(EROWS, 128)
    dst2d = jnp.concatenate(
        [dst, jnp.zeros((pad,), i32)]).reshape(EROWS, 128)
    batch3d = jnp.concatenate(
        [batch, jnp.full((NPOOL - N,), G, i32)]).reshape(NW, 3, 128)

    wcat1 = jnp.concatenate([W1.transpose(1, 0, 2).reshape(D, NREL * D),
                             root1], axis=1)
    wcat2 = jnp.concatenate([W2.transpose(1, 0, 2).reshape(D, NREL * D),
                             root2], axis=1)

    h1cat = _matmul1(x, wcat1)

    cnt32 = _counts(kd2d).reshape(NW, TBL)
    inv = _inv_counts(cnt32)
    w1d = _scales(inv, kd2d).reshape(EP)

    acc1 = _scatter_layer(h1cat.reshape(RK * N, D), ks2d, dst2d, w1d)
    acc1 = acc1.reshape(NC, NACC, D)

    h2cat = _comb_matmul(acc1[0, :N], acc1[1, :N],
                         h1cat[:, NREL * D:], b1.reshape(1, D), wcat2)
    acc2 = _scatter_layer(h2cat.reshape(RK * N, D), ks2d, dst2d, w1d)
    acc2 = acc2.reshape(NC, NACC, D)

    h2p = _comb_relu(acc2[0, :N], acc2[1, :N],
                     h2cat[:, NREL * D:], b2.reshape(1, D))

    pool2, cntb = _pool(h2p, batch3d)
    cntb = cntb.reshape(NW, PB)
    return _mlp_head(pool2, cntb, Wm1, bm1.reshape(1, MLP),
                     Wm2, bm2.reshape(1, NCLS))
